# baseline (speedup 1.0000x reference)
import functools

import jax
import jax.numpy as jnp
from jax import lax
from jax.experimental import pallas as pl
from jax.experimental.pallas import tpu as pltpu
from jax.experimental.pallas import tpu_sc as plsc

B, N, F, D = 64, 256, 9, 768
NC, NS, L = 2, 16, 16
NW = NC * NS
NCALL = 4
PN = N // NCALL
PPW = PN // NW
C = B
NSLOT = 2

_mesh = plsc.VectorSubcoreMesh(core_axis_name="c", subcore_axis_name="s")


def _make_body(tokens):
    tokoff = B if tokens else 0
    scratch = [
        pltpu.VMEM((PPW, F, C), jnp.int32),
        pltpu.VMEM((PPW, C), jnp.int32),
        pltpu.VMEM((PPW, C), jnp.int32),
        pltpu.VMEM((NSLOT, C, D), jnp.float32),
        [pltpu.SemaphoreType.DMA] * NSLOT,
        [pltpu.SemaphoreType.DMA] * NSLOT,
    ]
    if tokens:
        scratch.insert(4, pltpu.VMEM((2, D), jnp.float32))

    def body(x_hbm, ind_hbm, outd_hbm, atab, itab, otab, *rest):
        if tokens:
            tok, out_hbm, x_v, ind_v, outd_v, acc, tok_v, semg, semo = rest
        else:
            out_hbm, x_v, ind_v, outd_v, acc, semg, semo = rest
        w = lax.axis_index("s") * NC + lax.axis_index("c")

        pltpu.sync_copy(x_hbm.at[w], x_v)
        pltpu.sync_copy(ind_hbm.at[w], ind_v)
        pltpu.sync_copy(outd_hbm.at[w], outd_v)
        if tokens:
            pltpu.sync_copy(tok, tok_v.at[pl.ds(0, 1)])
            pltpu.sync_copy(tok, tok_v.at[pl.ds(1, 1)])
            pltpu.sync_copy(tok_v, out_hbm.at[pl.ds(2 * w, 2)])

        def zero_slot(p):
            z = jnp.zeros((L,), jnp.float32)

            @pl.loop(0, D // L)
            def _z(j):
                sl = pl.ds(j * L, L)
                for i in range(C):
                    acc[p, i, sl] = z

        def gathers(c, p):
            copies = [pltpu.make_async_copy(atab.at[x_v.at[c, f]], acc.at[p],
                                            semg[p]) for f in range(F)]
            copies.append(pltpu.make_async_copy(itab.at[ind_v.at[c]],
                                                acc.at[p], semg[p]))
            copies.append(pltpu.make_async_copy(otab.at[outd_v.at[c]],
                                                acc.at[p], semg[p]))
            return copies

        def fire_gathers(c, p):
            for f in range(F):
                pltpu.async_copy(atab.at[x_v.at[c, f]], acc.at[p], semg[p],
                                 add=True)
            pltpu.async_copy(itab.at[ind_v.at[c]], acc.at[p], semg[p],
                             add=True)
            pltpu.async_copy(otab.at[outd_v.at[c]], acc.at[p], semg[p],
                             add=True)

        def wait_gathers(c, p):
            for cp in gathers(c, p):
                cp.wait()

        def out_copy(c, p):
            row0 = tokoff + (w * PPW + c) * B
            return pltpu.make_async_copy(
                acc.at[p], out_hbm.at[pl.ds(row0, C)], semo[p])

        for p in range(NSLOT):
            zero_slot(p)
            fire_gathers(p, p)
        for p in range(NSLOT):
            wait_gathers(p, p)
            out_copy(p, p).start()
        for p in range(NSLOT):
            out_copy(p, p).wait()

    return functools.partial(
        pl.kernel,
        out_type=jax.ShapeDtypeStruct((tokoff + PN * B, D), jnp.float32),
        mesh=_mesh,
        compiler_params=pltpu.CompilerParams(use_tc_tiling_on_sc=False),
        scratch_types=scratch,
    )(body)


_body_tok = _make_body(True)
_body_plain = _make_body(False)


def kernel(x, in_degree, out_degree, atom_table, in_deg_table, out_deg_table,
           graph_token):
    xs = x.transpose(1, 2, 0)
    inds = in_degree.transpose(1, 0)
    outds = out_degree.transpose(1, 0)
    parts = []
    for j in range(NCALL):
        sl = slice(j * PN, (j + 1) * PN)
        xj = xs[sl].reshape(NW, PPW, F, C)
        ij = inds[sl].reshape(NW, PPW, C)
        oj = outds[sl].reshape(NW, PPW, C)
        if j == 0:
            parts.append(_body_tok(xj, ij, oj, atom_table, in_deg_table,
                                   out_deg_table, graph_token))
        else:
            parts.append(_body_plain(xj, ij, oj, atom_table, in_deg_table,
                                     out_deg_table))
    out = jnp.concatenate(parts, axis=0)
    return out.reshape(N + 1, B, D).transpose(1, 0, 2)
